# Initial kernel scaffold; baseline (speedup 1.0000x reference)
import jax
import jax.numpy as jnp
from jax.experimental import pallas as pl


def kernel(feats, edge_index, W_self1, W_neigh1, b1, W_self2, W_neigh2, b2):
    raise NotImplementedError("write your pallas kernel here")


# speedup vs baseline: 5.3967x; 5.3967x over previous
import jax
import jax.numpy as jnp
from jax import lax
from jax.experimental import pallas as pl
from jax.experimental.pallas import tpu as pltpu
from jax.experimental.pallas import tpu_sc as plsc

N = 10000
D = 128
E = 320000

NC = 2
NS = 16
NW = NC * NS
EPW = E // NW
CH = 80
NCHUNK = EPW // CH
RPT = 640
NPAD = RPT * NS


def _make_sc_agg(with_deg: bool):
    mesh = plsc.VectorSubcoreMesh(core_axis_name="c", subcore_axis_name="s")
    out_type = [jax.ShapeDtypeStruct((NC, NPAD, D), jnp.float32)]
    scratch_types = [
        pltpu.VMEM((CH,), jnp.int32),
        pltpu.VMEM((CH,), jnp.int32),
        pltpu.VMEM((CH, D), jnp.float32),
        pltpu.VMEM_SHARED((NPAD, D), jnp.float32),
        pltpu.SemaphoreType.DMA,
    ]
    if with_deg:
        out_type.append(jax.ShapeDtypeStruct((NW, NPAD), jnp.float32))
        scratch_types.append(pltpu.VMEM((NPAD,), jnp.float32))

    def body(*refs):
        if with_deg:
            (feats, src, dst, zfeat,
             agg_out, deg_out,
             sidx, didx, rows, agg_sh, sem, deg_v) = refs
        else:
            (feats, src, dst, zfeat,
             agg_out,
             sidx, didx, rows, agg_sh, sem) = refs
        c = lax.axis_index("c")
        s = lax.axis_index("s")
        wid = s * NC + c
        base = wid * EPW
        r0 = s * RPT

        def zinit(j, carry):
            rr = r0 + j * CH
            pltpu.sync_copy(zfeat.at[pl.ds(rr, CH)], rows)
            pltpu.sync_copy(rows, agg_sh.at[pl.ds(rr, CH)])
            return carry

        lax.fori_loop(0, RPT // CH, zinit, 0)
        if with_deg:
            def zdeg(j, carry):
                deg_v[pl.ds(j * 16, 16)] = jnp.zeros((16,), jnp.float32)
                return carry

            lax.fori_loop(0, NPAD // 16, zdeg, 0)
        plsc.subcore_barrier()

        ones16 = jnp.full((16,), 1.0, jnp.float32)

        def chunk(i, carry):
            off = base + i * CH
            pltpu.sync_copy(src.at[pl.ds(off, CH)], sidx)
            pltpu.sync_copy(dst.at[pl.ds(off, CH)], didx)
            pltpu.async_copy(feats.at[sidx], rows, sem).wait()
            pltpu.sync_copy(rows, agg_sh.at[didx], add=True)
            if with_deg:
                for j in range(CH // 16):
                    dv = didx[pl.ds(j * 16, 16)]
                    plsc.addupdate_scatter(deg_v, [dv], ones16)
            return carry

        lax.fori_loop(0, NCHUNK, chunk, 0)
        plsc.subcore_barrier()

        def wout(j, carry):
            rr = r0 + j * CH
            pltpu.sync_copy(agg_sh.at[pl.ds(rr, CH)], rows)
            pltpu.sync_copy(rows, agg_out.at[c, pl.ds(rr, CH)])
            return carry

        lax.fori_loop(0, RPT // CH, wout, 0)
        if with_deg:
            pltpu.sync_copy(deg_v, deg_out.at[wid])

    return pl.kernel(
        body, out_type=out_type, mesh=mesh, scratch_types=scratch_types,
        compiler_params=pltpu.CompilerParams(needs_layout_passes=False))


_sc_agg_deg = _make_sc_agg(with_deg=True)
_sc_agg = _make_sc_agg(with_deg=False)

BR = 1024


def _tc_combine(h, aggp, degT, W_self, W_neigh, b):
    def body(h_ref, aggp_ref, degT_ref, ws_ref, wn_ref, b_ref, out_ref):
        agg = aggp_ref[0] + aggp_ref[1]
        deg = jnp.sum(degT_ref[...], axis=1, keepdims=True)
        hn = agg / jnp.maximum(deg, 1.0)
        out_ref[...] = (
            jnp.dot(h_ref[...], ws_ref[...], preferred_element_type=jnp.float32)
            + jnp.dot(hn, wn_ref[...], preferred_element_type=jnp.float32)
            + b_ref[...]
        )

    return pl.pallas_call(
        body,
        grid=(NPAD // BR,),
        in_specs=[
            pl.BlockSpec((BR, D), lambda i: (i, 0)),
            pl.BlockSpec((NC, BR, D), lambda i: (0, i, 0)),
            pl.BlockSpec((BR, 128), lambda i: (i, 0)),
            pl.BlockSpec((D, D), lambda i: (0, 0)),
            pl.BlockSpec((D, D), lambda i: (0, 0)),
            pl.BlockSpec((1, D), lambda i: (0, 0)),
        ],
        out_specs=pl.BlockSpec((BR, D), lambda i: (i, 0)),
        out_shape=jax.ShapeDtypeStruct((NPAD, D), jnp.float32),
    )(h, aggp, degT, W_self, W_neigh, b.reshape(1, D))


def kernel(feats, edge_index, W_self1, W_neigh1, b1, W_self2, W_neigh2, b2):
    src = edge_index[0].astype(jnp.int32)
    dst = edge_index[1].astype(jnp.int32)
    zfeat = jnp.zeros((NPAD, D), jnp.float32)
    featsP = zfeat.at[:N].set(feats)

    aggp1, degp = _sc_agg_deg(featsP, src, dst, zfeat)
    degT = jnp.zeros((NPAD, 128), jnp.float32).at[:, :NW].set(degp.T)
    h1 = _tc_combine(featsP, aggp1, degT, W_self1, W_neigh1, b1)
    (aggp2,) = _sc_agg(h1, src, dst, zfeat)
    out = _tc_combine(h1, aggp2, degT, W_self2, W_neigh2, b2)
    return out[:N]


# speedup vs baseline: 11.0414x; 2.0459x over previous
import jax
import jax.numpy as jnp
from jax import lax
from jax.experimental import pallas as pl
from jax.experimental.pallas import tpu as pltpu
from jax.experimental.pallas import tpu_sc as plsc

N = 10000
D = 128
E = 320000

NC = 2
NS = 16
NW = NC * NS
EPW = E // NW
CH = 128
NB = EPW // CH
TAIL = EPW - NB * CH
EP = E + 4 * CH
RPT = 640
NPAD = RPT * NS


def _make_sc_agg(with_deg: bool):
    mesh = plsc.VectorSubcoreMesh(core_axis_name="c", subcore_axis_name="s")
    out_type = [jax.ShapeDtypeStruct((NC, NPAD, D), jnp.float32)]
    scratch_types = [
        pltpu.VMEM((CH,), jnp.int32),
        pltpu.VMEM((CH,), jnp.int32),
        pltpu.VMEM((CH,), jnp.int32),
        pltpu.VMEM((CH,), jnp.int32),
        pltpu.VMEM((CH, D), jnp.float32),
        pltpu.VMEM((CH, D), jnp.float32),
        pltpu.VMEM((TAIL,), jnp.int32),
        pltpu.VMEM((TAIL,), jnp.int32),
        pltpu.VMEM((TAIL, D), jnp.float32),
        pltpu.VMEM_SHARED((NPAD, D), jnp.float32),
        pltpu.SemaphoreType.DMA,
        pltpu.SemaphoreType.DMA,
        pltpu.SemaphoreType.DMA,
        pltpu.SemaphoreType.DMA,
        pltpu.SemaphoreType.DMA,
        pltpu.SemaphoreType.DMA,
        pltpu.SemaphoreType.DMA,
    ]
    if with_deg:
        out_type.append(jax.ShapeDtypeStruct((NW, NPAD), jnp.float32))
        scratch_types.append(pltpu.VMEM((NPAD,), jnp.float32))

    def body(*refs):
        if with_deg:
            (feats, srcr, dstr, zfeat,
             agg_out, deg_out,
             sidx0, sidx1, didx0, didx1, rows0, rows1, sidxT, didxT, rowsT,
             agg_sh, sg0, sg1, si0, si1, sd0, sd1, sT, deg_v) = refs
        else:
            (feats, srcr, dstr, zfeat,
             agg_out,
             sidx0, sidx1, didx0, didx1, rows0, rows1, sidxT, didxT, rowsT,
             agg_sh, sg0, sg1, si0, si1, sd0, sd1, sT) = refs
        c = lax.axis_index("c")
        s = lax.axis_index("s")
        wid = s * NC + c
        base = wid * EPW
        r0 = s * RPT

        def zinit(j, carry):
            rr = r0 + j * CH
            pltpu.sync_copy(zfeat.at[pl.ds(rr, CH)], rows0)
            pltpu.sync_copy(rows0, agg_sh.at[pl.ds(rr, CH)])
            return carry

        lax.fori_loop(0, RPT // CH, zinit, 0)
        if with_deg:
            def zdeg(j, carry):
                deg_v[pl.ds(j * 16, 16)] = jnp.zeros((16,), jnp.float32)
                return carry

            lax.fori_loop(0, NPAD // 16, zdeg, 0)

        ones16 = jnp.full((16,), 1.0, jnp.float32)

        def start_idx(ck, sidx_b, didx_b, si_b, sd_b):
            off = base + ck * CH
            pltpu.async_copy(srcr.at[pl.ds(off, CH)], sidx_b, si_b)
            pltpu.async_copy(dstr.at[pl.ds(off, CH)], didx_b, sd_b)

        def wait_idx(sidx_b, didx_b, si_b, sd_b):
            pltpu.make_async_copy(srcr.at[pl.ds(0, CH)], sidx_b, si_b).wait()
            pltpu.make_async_copy(dstr.at[pl.ds(0, CH)], didx_b, sd_b).wait()

        def wait_gather(rows_b, sg_b):
            pltpu.make_async_copy(feats.at[pl.ds(0, CH)], rows_b, sg_b).wait()

        def step(ck, sA, dA, rA, siA, sdA, sgA, sB, dB, rB, siB, sdB, sgB):
            wait_gather(rA, sgA)
            wait_idx(sB, dB, siB, sdB)
            pltpu.async_copy(feats.at[sB], rB, sgB)
            pltpu.sync_copy(rA, agg_sh.at[dA], add=True)
            if with_deg:
                for j in range(CH // 16):
                    dv = dA[pl.ds(j * 16, 16)]
                    plsc.addupdate_scatter(deg_v, [dv], ones16)
            start_idx(ck + 2, sA, dA, siA, sdA)

        start_idx(0, sidx0, didx0, si0, sd0)
        wait_idx(sidx0, didx0, si0, sd0)
        pltpu.async_copy(feats.at[sidx0], rows0, sg0)
        start_idx(1, sidx1, didx1, si1, sd1)

        def pair(p, carry):
            ck = 2 * p
            step(ck, sidx0, didx0, rows0, si0, sd0, sg0,
                 sidx1, didx1, rows1, si1, sd1, sg1)
            step(ck + 1, sidx1, didx1, rows1, si1, sd1, sg1,
                 sidx0, didx0, rows0, si0, sd0, sg0)
            return carry

        lax.fori_loop(0, NB // 2, pair, 0)

        wait_gather(rows0, sg0)
        wait_idx(sidx1, didx1, si1, sd1)

        offT = base + NB * CH
        pltpu.sync_copy(srcr.at[pl.ds(offT, TAIL)], sidxT)
        pltpu.sync_copy(dstr.at[pl.ds(offT, TAIL)], didxT)
        pltpu.async_copy(feats.at[sidxT], rowsT, sT).wait()
        pltpu.sync_copy(rowsT, agg_sh.at[didxT], add=True)
        if with_deg:
            plsc.addupdate_scatter(deg_v, [didxT[...]], ones16)

        plsc.subcore_barrier()

        def wout(j, carry):
            rr = r0 + j * CH
            pltpu.sync_copy(agg_sh.at[pl.ds(rr, CH)], rows0)
            pltpu.sync_copy(rows0, agg_out.at[c, pl.ds(rr, CH)])
            return carry

        lax.fori_loop(0, RPT // CH, wout, 0)
        if with_deg:
            pltpu.sync_copy(deg_v, deg_out.at[wid])

    return pl.kernel(
        body, out_type=out_type, mesh=mesh, scratch_types=scratch_types,
        compiler_params=pltpu.CompilerParams(needs_layout_passes=False))


_sc_agg_deg = _make_sc_agg(with_deg=True)
_sc_agg = _make_sc_agg(with_deg=False)

BR = 1024


def _tc_combine(h, aggp, degT, W_self, W_neigh, b):
    def body(h_ref, aggp_ref, degT_ref, ws_ref, wn_ref, b_ref, out_ref):
        agg = aggp_ref[0] + aggp_ref[1]
        deg = jnp.sum(degT_ref[...], axis=1, keepdims=True)
        hn = agg / jnp.maximum(deg, 1.0)
        out_ref[...] = (
            jnp.dot(h_ref[...], ws_ref[...], preferred_element_type=jnp.float32)
            + jnp.dot(hn, wn_ref[...], preferred_element_type=jnp.float32)
            + b_ref[...]
        )

    return pl.pallas_call(
        body,
        grid=(NPAD // BR,),
        in_specs=[
            pl.BlockSpec((BR, D), lambda i: (i, 0)),
            pl.BlockSpec((NC, BR, D), lambda i: (0, i, 0)),
            pl.BlockSpec((BR, 128), lambda i: (i, 0)),
            pl.BlockSpec((D, D), lambda i: (0, 0)),
            pl.BlockSpec((D, D), lambda i: (0, 0)),
            pl.BlockSpec((1, D), lambda i: (0, 0)),
        ],
        out_specs=pl.BlockSpec((BR, D), lambda i: (i, 0)),
        out_shape=jax.ShapeDtypeStruct((NPAD, D), jnp.float32),
    )(h, aggp, degT, W_self, W_neigh, b.reshape(1, D))


def kernel(feats, edge_index, W_self1, W_neigh1, b1, W_self2, W_neigh2, b2):
    src = edge_index[0].astype(jnp.int32)
    dst = edge_index[1].astype(jnp.int32)
    srcP = jnp.zeros((EP,), jnp.int32).at[:E].set(src)
    dstP = jnp.zeros((EP,), jnp.int32).at[:E].set(dst)
    zfeat = jnp.zeros((NPAD, D), jnp.float32)
    featsP = zfeat.at[:N].set(feats)

    aggp1, degp = _sc_agg_deg(featsP, srcP, dstP, zfeat)
    degT = jnp.zeros((NPAD, 128), jnp.float32).at[:, :NW].set(degp.T)
    h1 = _tc_combine(featsP, aggp1, degT, W_self1, W_neigh1, b1)
    (aggp2,) = _sc_agg(h1, srcP, dstP, zfeat)
    out = _tc_combine(h1, aggp2, degT, W_self2, W_neigh2, b2)
    return out[:N]


# speedup vs baseline: 11.3716x; 1.0299x over previous
import jax
import jax.numpy as jnp
from jax import lax
from jax.experimental import pallas as pl
from jax.experimental.pallas import tpu as pltpu
from jax.experimental.pallas import tpu_sc as plsc

N = 10000
D = 128
E = 320000

NC = 2
NS = 16
NW = NC * NS
EPW = E // NW
CH = 128
NB = EPW // CH
TAIL = EPW - NB * CH
RPT = 640
NPAD = RPT * NS


def _make_sc_agg(with_deg: bool):
    mesh = plsc.VectorSubcoreMesh(core_axis_name="c", subcore_axis_name="s")
    out_type = [jax.ShapeDtypeStruct((NC, NPAD, D), jnp.float32)]
    scratch_types = [
        pltpu.VMEM((CH,), jnp.int32),
        pltpu.VMEM((CH,), jnp.int32),
        pltpu.VMEM((CH,), jnp.int32),
        pltpu.VMEM((CH,), jnp.int32),
        pltpu.VMEM((CH, D), jnp.float32),
        pltpu.VMEM((CH, D), jnp.float32),
        pltpu.VMEM((TAIL,), jnp.int32),
        pltpu.VMEM((TAIL,), jnp.int32),
        pltpu.VMEM((TAIL, D), jnp.float32),
        pltpu.VMEM_SHARED((NPAD, D), jnp.float32),
        pltpu.SemaphoreType.DMA,
        pltpu.SemaphoreType.DMA,
        pltpu.SemaphoreType.DMA,
        pltpu.SemaphoreType.DMA,
        pltpu.SemaphoreType.DMA,
        pltpu.SemaphoreType.DMA,
        pltpu.SemaphoreType.DMA,
    ]
    if with_deg:
        out_type.append(jax.ShapeDtypeStruct((NW, NPAD), jnp.float32))
        scratch_types.append(pltpu.VMEM((NPAD,), jnp.float32))

    def body(*refs):
        if with_deg:
            (feats, srcr, dstr, zfeat,
             agg_out, deg_out,
             sidx0, sidx1, didx0, didx1, rows0, rows1, sidxT, didxT, rowsT,
             agg_sh, sg0, sg1, si0, si1, sd0, sd1, sT, deg_v) = refs
        else:
            (feats, srcr, dstr, zfeat,
             agg_out,
             sidx0, sidx1, didx0, didx1, rows0, rows1, sidxT, didxT, rowsT,
             agg_sh, sg0, sg1, si0, si1, sd0, sd1, sT) = refs
        c = lax.axis_index("c")
        s = lax.axis_index("s")
        wid = s * NC + c
        base = wid * EPW
        r0 = s * RPT

        def zinit(j, carry):
            rr = r0 + j * CH
            pltpu.sync_copy(zfeat.at[pl.ds(rr, CH)], rows0)
            pltpu.sync_copy(rows0, agg_sh.at[pl.ds(rr, CH)])
            return carry

        lax.fori_loop(0, RPT // CH, zinit, 0)
        if with_deg:
            def zdeg(j, carry):
                deg_v[pl.ds(j * 16, 16)] = jnp.zeros((16,), jnp.float32)
                return carry

            lax.fori_loop(0, NPAD // 16, zdeg, 0)

        ones16 = jnp.full((16,), 1.0, jnp.float32)

        def start_idx(ck, sidx_b, didx_b, si_b, sd_b):
            off = jnp.minimum(base + ck * CH, E - CH)
            pltpu.async_copy(srcr.at[pl.ds(off, CH)], sidx_b, si_b)
            pltpu.async_copy(dstr.at[pl.ds(off, CH)], didx_b, sd_b)

        def wait_idx(sidx_b, didx_b, si_b, sd_b):
            pltpu.make_async_copy(srcr.at[pl.ds(0, CH)], sidx_b, si_b).wait()
            pltpu.make_async_copy(dstr.at[pl.ds(0, CH)], didx_b, sd_b).wait()

        def wait_gather(rows_b, sg_b):
            pltpu.make_async_copy(feats.at[pl.ds(0, CH)], rows_b, sg_b).wait()

        def step(ck, sA, dA, rA, siA, sdA, sgA, sB, dB, rB, siB, sdB, sgB):
            wait_gather(rA, sgA)
            wait_idx(sB, dB, siB, sdB)
            pltpu.async_copy(feats.at[sB], rB, sgB)
            pltpu.sync_copy(rA, agg_sh.at[dA], add=True)
            if with_deg:
                for j in range(CH // 16):
                    dv = dA[pl.ds(j * 16, 16)]
                    plsc.addupdate_scatter(deg_v, [dv], ones16)
            start_idx(ck + 2, sA, dA, siA, sdA)

        start_idx(0, sidx0, didx0, si0, sd0)
        wait_idx(sidx0, didx0, si0, sd0)
        pltpu.async_copy(feats.at[sidx0], rows0, sg0)
        start_idx(1, sidx1, didx1, si1, sd1)

        def pair(p, carry):
            ck = 2 * p
            step(ck, sidx0, didx0, rows0, si0, sd0, sg0,
                 sidx1, didx1, rows1, si1, sd1, sg1)
            step(ck + 1, sidx1, didx1, rows1, si1, sd1, sg1,
                 sidx0, didx0, rows0, si0, sd0, sg0)
            return carry

        lax.fori_loop(0, NB // 2, pair, 0)

        wait_gather(rows0, sg0)
        wait_idx(sidx1, didx1, si1, sd1)

        offT = base + NB * CH
        pltpu.sync_copy(srcr.at[pl.ds(offT, TAIL)], sidxT)
        pltpu.sync_copy(dstr.at[pl.ds(offT, TAIL)], didxT)
        pltpu.async_copy(feats.at[sidxT], rowsT, sT).wait()
        pltpu.sync_copy(rowsT, agg_sh.at[didxT], add=True)
        if with_deg:
            plsc.addupdate_scatter(deg_v, [didxT[...]], ones16)

        plsc.subcore_barrier()

        def wout(j, carry):
            rr = r0 + j * CH
            pltpu.sync_copy(agg_sh.at[pl.ds(rr, CH)], rows0)
            pltpu.sync_copy(rows0, agg_out.at[c, pl.ds(rr, CH)])
            return carry

        lax.fori_loop(0, RPT // CH, wout, 0)
        if with_deg:
            pltpu.sync_copy(deg_v, deg_out.at[wid])

    return pl.kernel(
        body, out_type=out_type, mesh=mesh, scratch_types=scratch_types,
        compiler_params=pltpu.CompilerParams(needs_layout_passes=False))


_sc_agg_deg = _make_sc_agg(with_deg=True)
_sc_agg = _make_sc_agg(with_deg=False)

BR = 1024


def _tc_combine(h, aggp, degT, W_self, W_neigh, b):
    def body(h_ref, aggp_ref, degT_ref, ws_ref, wn_ref, b_ref, out_ref):
        agg = aggp_ref[0] + aggp_ref[1]
        deg = jnp.sum(degT_ref[...], axis=1, keepdims=True)
        hn = agg / jnp.maximum(deg, 1.0)
        out_ref[...] = (
            jnp.dot(h_ref[...], ws_ref[...], preferred_element_type=jnp.float32)
            + jnp.dot(hn, wn_ref[...], preferred_element_type=jnp.float32)
            + b_ref[...]
        )

    return pl.pallas_call(
        body,
        grid=(NPAD // BR,),
        in_specs=[
            pl.BlockSpec((BR, D), lambda i: (i, 0)),
            pl.BlockSpec((NC, BR, D), lambda i: (0, i, 0)),
            pl.BlockSpec((BR, 128), lambda i: (i, 0)),
            pl.BlockSpec((D, D), lambda i: (0, 0)),
            pl.BlockSpec((D, D), lambda i: (0, 0)),
            pl.BlockSpec((1, D), lambda i: (0, 0)),
        ],
        out_specs=pl.BlockSpec((BR, D), lambda i: (i, 0)),
        out_shape=jax.ShapeDtypeStruct((N, D), jnp.float32),
    )(h, aggp, degT, W_self, W_neigh, b.reshape(1, D))


def kernel(feats, edge_index, W_self1, W_neigh1, b1, W_self2, W_neigh2, b2):
    src = edge_index[0].astype(jnp.int32)
    dst = edge_index[1].astype(jnp.int32)
    zfeat = jnp.zeros((NPAD, D), jnp.float32)

    aggp1, degp = _sc_agg_deg(feats, src, dst, zfeat)
    degT = jnp.zeros((NPAD, 128), jnp.float32).at[:, :NW].set(degp.T)
    h1 = _tc_combine(feats, aggp1, degT, W_self1, W_neigh1, b1)
    (aggp2,) = _sc_agg(h1, src, dst, zfeat)
    return _tc_combine(h1, aggp2, degT, W_self2, W_neigh2, b2)
